# baseline (speedup 1.0000x reference)
import functools
import math

import jax
import jax.numpy as jnp
from jax.experimental import pallas as pl
from jax.experimental.pallas import tpu as pltpu


def _qkv_kernel(x_ref, wq_ref, wk_ref, wv_ref, b_ref, out_ref):
    x = x_ref[...]
    d = x.shape[1]
    q = jnp.dot(x, wq_ref[...], preferred_element_type=jnp.float32)
    k = jnp.dot(x, wk_ref[...], preferred_element_type=jnp.float32)
    v = jnp.dot(x, wv_ref[...], preferred_element_type=jnp.float32)
    b = b_ref[...]
    out_ref[:, 0:d] = (q + b[:, 0:d]).astype(jnp.bfloat16)
    out_ref[:, d:2 * d] = (k + b[:, d:2 * d]).astype(jnp.bfloat16)
    out_ref[:, 2 * d:3 * d] = (v + b[:, 2 * d:3 * d]).astype(jnp.bfloat16)


def _attn_kernel(q_ref, k_ref, v_ref, x_ref, wo_ref, bo_ref, xa_ref, *, scale):
    q = q_ref[...]
    k = k_ref[...]
    s = jax.lax.dot_general(q, k, (((1,), (1,)), ((), ())),
                            preferred_element_type=jnp.float32) * scale
    m = jnp.max(s, axis=1, keepdims=True)
    p = jnp.exp(s - m)
    p = p / jnp.sum(p, axis=1, keepdims=True)
    attn = jnp.dot(p.astype(jnp.bfloat16), v_ref[...],
                   preferred_element_type=jnp.float32)
    o = jnp.dot(attn, wo_ref[...],
                preferred_element_type=jnp.float32) + bo_ref[...]
    xa_ref[...] = x_ref[...] + o


def _moe_kernel(xa_ref, wr_ref, w1_ref, b1_ref, w2_ref, b2_ref, out_ref,
                gates_ref, *, n_chunks):
    e = pl.program_id(1)

    @pl.when(e == 0)
    def _():
        xa = xa_ref[...]
        logits = jnp.dot(xa, wr_ref[...], preferred_element_type=jnp.float32)
        lmax = jnp.max(logits, axis=1, keepdims=True)
        ex = jnp.exp(logits - lmax)
        gates_ref[...] = ex / jnp.sum(ex, axis=1, keepdims=True)
        out_ref[...] = xa

    gates = gates_ref[...]
    onehot = (jax.lax.broadcasted_iota(jnp.int32, (1, gates.shape[1]), 1) == e)
    g = jnp.sum(gates * onehot, axis=1, keepdims=True)

    xa = xa_ref[...]
    fc = w1_ref.shape[2] // n_chunks
    part = None
    for c in range(n_chunks):
        sl = slice(c * fc, (c + 1) * fc)
        h = jnp.dot(xa, w1_ref[0, :, sl], preferred_element_type=jnp.float32)
        h = jnp.maximum(h + b1_ref[0, :, sl], 0.0)
        p = jnp.dot(h, w2_ref[0, sl, :], preferred_element_type=jnp.float32)
        part = p if part is None else part + p
    out_ref[...] += g * (part + b2_ref[0])


def kernel(x, Wq, bq, Wk, bk, Wv, bv, Wo, bo, Wr, W1, b1, W2, b2):
    B, S, D = x.shape
    E, _, F = W1.shape
    xf = x.reshape(S, D)
    scale = 1.0 / math.sqrt(D)

    bqkv = jnp.concatenate([bq, bk, bv]).reshape(1, 3 * D)
    Tq = min(512, S)
    qkv = pl.pallas_call(
        _qkv_kernel,
        grid=(S // Tq,),
        in_specs=[
            pl.BlockSpec((Tq, D), lambda i: (i, 0)),
            pl.BlockSpec((D, D), lambda i: (0, 0)),
            pl.BlockSpec((D, D), lambda i: (0, 0)),
            pl.BlockSpec((D, D), lambda i: (0, 0)),
            pl.BlockSpec((1, 3 * D), lambda i: (0, 0)),
        ],
        out_specs=pl.BlockSpec((Tq, 3 * D), lambda i: (i, 0)),
        out_shape=jax.ShapeDtypeStruct((S, 3 * D), jnp.bfloat16),
        compiler_params=pltpu.CompilerParams(
            dimension_semantics=("parallel",)),
    )(xf, Wq, Wk, Wv, bqkv)

    Ta = min(512, S)
    xa = pl.pallas_call(
        functools.partial(_attn_kernel, scale=scale),
        grid=(S // Ta,),
        in_specs=[
            pl.BlockSpec((Ta, D), lambda i: (i, 0)),
            pl.BlockSpec((S, D), lambda i: (0, 1)),
            pl.BlockSpec((S, D), lambda i: (0, 2)),
            pl.BlockSpec((Ta, D), lambda i: (i, 0)),
            pl.BlockSpec((D, D), lambda i: (0, 0)),
            pl.BlockSpec((1, D), lambda i: (0, 0)),
        ],
        out_specs=pl.BlockSpec((Ta, D), lambda i: (i, 0)),
        out_shape=jax.ShapeDtypeStruct((S, D), jnp.float32),
        compiler_params=pltpu.CompilerParams(
            dimension_semantics=("parallel",)),
    )(qkv, qkv, qkv, xf, Wo, bo.reshape(1, D))

    Tm = min(1024, S)
    b1r = b1.reshape(E, 1, F)
    b2r = b2.reshape(E, 1, D)
    out = pl.pallas_call(
        functools.partial(_moe_kernel, n_chunks=4),
        grid=(S // Tm, E),
        in_specs=[
            pl.BlockSpec((Tm, D), lambda t, e: (t, 0)),
            pl.BlockSpec((D, E), lambda t, e: (0, 0)),
            pl.BlockSpec((1, D, F), lambda t, e: (e, 0, 0)),
            pl.BlockSpec((1, 1, F), lambda t, e: (e, 0, 0)),
            pl.BlockSpec((1, F, D), lambda t, e: (e, 0, 0)),
            pl.BlockSpec((1, 1, D), lambda t, e: (e, 0, 0)),
        ],
        out_specs=pl.BlockSpec((Tm, D), lambda t, e: (t, 0)),
        out_shape=jax.ShapeDtypeStruct((S, D), jnp.float32),
        scratch_shapes=[
            pltpu.VMEM((Tm, E), jnp.float32),
        ],
        compiler_params=pltpu.CompilerParams(
            dimension_semantics=("parallel", "arbitrary")),
    )(xa, Wr, W1, b1r, W2, b2r)

    return out.reshape(B, S, D)


# speedup vs baseline: 1.0013x; 1.0013x over previous
import functools
import math

import jax
import jax.numpy as jnp
from jax.experimental import pallas as pl
from jax.experimental.pallas import tpu as pltpu


def _qkv_kernel(x_ref, wq_ref, wk_ref, wv_ref, b_ref, out_ref):
    x = x_ref[...]
    d = x.shape[1]
    q = jnp.dot(x, wq_ref[...], preferred_element_type=jnp.float32)
    k = jnp.dot(x, wk_ref[...], preferred_element_type=jnp.float32)
    v = jnp.dot(x, wv_ref[...], preferred_element_type=jnp.float32)
    b = b_ref[...]
    out_ref[:, 0:d] = (q + b[:, 0:d]).astype(jnp.bfloat16)
    out_ref[:, d:2 * d] = (k + b[:, d:2 * d]).astype(jnp.bfloat16)
    out_ref[:, 2 * d:3 * d] = (v + b[:, 2 * d:3 * d]).astype(jnp.bfloat16)


def _attn_kernel(q_ref, k_ref, v_ref, x_ref, wo_ref, bo_ref, xa_ref, *, scale):
    q = q_ref[...]
    k = k_ref[...]
    s = jax.lax.dot_general(q, k, (((1,), (1,)), ((), ())),
                            preferred_element_type=jnp.float32) * scale
    m = jnp.max(s, axis=1, keepdims=True)
    p = jnp.exp(s - m)
    p = p / jnp.sum(p, axis=1, keepdims=True)
    attn = jnp.dot(p.astype(jnp.bfloat16), v_ref[...],
                   preferred_element_type=jnp.float32)
    o = jnp.dot(attn, wo_ref[...],
                preferred_element_type=jnp.float32) + bo_ref[...]
    xa_ref[...] = x_ref[...] + o


def _moe_kernel(xa_ref, wr_ref, w1_ref, b1_ref, w2_ref, b2_ref, out_ref,
                gates_ref, *, n_chunks):
    s = pl.program_id(1)
    e = s // 2
    half = s % 2

    @pl.when(s == 0)
    def _():
        xa = xa_ref[...]
        logits = jnp.dot(xa, wr_ref[...], preferred_element_type=jnp.float32)
        lmax = jnp.max(logits, axis=1, keepdims=True)
        ex = jnp.exp(logits - lmax)
        gates_ref[...] = ex / jnp.sum(ex, axis=1, keepdims=True)
        out_ref[...] = xa

    gates = gates_ref[...]
    onehot = (jax.lax.broadcasted_iota(jnp.int32, (1, gates.shape[1]), 1) == e)
    g = jnp.sum(gates * onehot, axis=1, keepdims=True)

    xa = xa_ref[...]
    fc = w1_ref.shape[2] // n_chunks
    part = None
    for c in range(n_chunks):
        sl = slice(c * fc, (c + 1) * fc)
        h = jnp.dot(xa, w1_ref[0, :, sl], preferred_element_type=jnp.float32)
        h = jnp.maximum(h + b1_ref[0, :, sl], 0.0)
        p = jnp.dot(h, w2_ref[0, sl, :], preferred_element_type=jnp.float32)
        part = p if part is None else part + p
    b2term = b2_ref[0] * (half == 0).astype(jnp.float32)
    out_ref[...] += g * (part + b2term)


def kernel(x, Wq, bq, Wk, bk, Wv, bv, Wo, bo, Wr, W1, b1, W2, b2):
    B, S, D = x.shape
    E, _, F = W1.shape
    xf = x.reshape(S, D)
    scale = 1.0 / math.sqrt(D)

    bqkv = jnp.concatenate([bq, bk, bv]).reshape(1, 3 * D)
    Tq = min(512, S)
    qkv = pl.pallas_call(
        _qkv_kernel,
        grid=(S // Tq,),
        in_specs=[
            pl.BlockSpec((Tq, D), lambda i: (i, 0)),
            pl.BlockSpec((D, D), lambda i: (0, 0)),
            pl.BlockSpec((D, D), lambda i: (0, 0)),
            pl.BlockSpec((D, D), lambda i: (0, 0)),
            pl.BlockSpec((1, 3 * D), lambda i: (0, 0)),
        ],
        out_specs=pl.BlockSpec((Tq, 3 * D), lambda i: (i, 0)),
        out_shape=jax.ShapeDtypeStruct((S, 3 * D), jnp.bfloat16),
        compiler_params=pltpu.CompilerParams(
            dimension_semantics=("parallel",)),
    )(xf, Wq, Wk, Wv, bqkv)

    Ta = min(512, S)
    xa = pl.pallas_call(
        functools.partial(_attn_kernel, scale=scale),
        grid=(S // Ta,),
        in_specs=[
            pl.BlockSpec((Ta, D), lambda i: (i, 0)),
            pl.BlockSpec((S, D), lambda i: (0, 1)),
            pl.BlockSpec((S, D), lambda i: (0, 2)),
            pl.BlockSpec((Ta, D), lambda i: (i, 0)),
            pl.BlockSpec((D, D), lambda i: (0, 0)),
            pl.BlockSpec((1, D), lambda i: (0, 0)),
        ],
        out_specs=pl.BlockSpec((Ta, D), lambda i: (i, 0)),
        out_shape=jax.ShapeDtypeStruct((S, D), jnp.float32),
        compiler_params=pltpu.CompilerParams(
            dimension_semantics=("parallel",)),
    )(qkv, qkv, qkv, xf, Wo, bo.reshape(1, D))

    Tm = min(2048, S)
    b1r = b1.reshape(E, 1, F)
    b2r = b2.reshape(E, 1, D)
    out = pl.pallas_call(
        functools.partial(_moe_kernel, n_chunks=2),
        grid=(S // Tm, 2 * E),
        in_specs=[
            pl.BlockSpec((Tm, D), lambda t, s: (t, 0)),
            pl.BlockSpec((D, E), lambda t, s: (0, 0)),
            pl.BlockSpec((1, D, F // 2),
                         lambda t, s: (s // 2, 0, s % 2)),
            pl.BlockSpec((1, 1, F // 2),
                         lambda t, s: (s // 2, 0, s % 2)),
            pl.BlockSpec((1, F // 2, D),
                         lambda t, s: (s // 2, s % 2, 0)),
            pl.BlockSpec((1, 1, D), lambda t, s: (s // 2, 0, 0)),
        ],
        out_specs=pl.BlockSpec((Tm, D), lambda t, s: (t, 0)),
        out_shape=jax.ShapeDtypeStruct((S, D), jnp.float32),
        scratch_shapes=[
            pltpu.VMEM((Tm, E), jnp.float32),
        ],
        compiler_params=pltpu.CompilerParams(
            dimension_semantics=("parallel", "arbitrary")),
    )(xa, Wr, W1, b1r, W2, b2r)

    return out.reshape(B, S, D)


# speedup vs baseline: 1.0215x; 1.0202x over previous
import functools
import math

import jax
import jax.numpy as jnp
from jax.experimental import pallas as pl
from jax.experimental.pallas import tpu as pltpu


def _attn_fused_kernel(x_ref, wq_ref, wk_ref, wv_ref, b_ref, wo_ref, bo_ref,
                       xa_ref, qkv_ref, *, n_tiles, scale):
    i = pl.program_id(0)
    t = x_ref.shape[0]
    d = x_ref.shape[1]

    @pl.when(i < n_tiles)
    def _():
        x = x_ref[...]
        b = b_ref[...]
        q = jnp.dot(x, wq_ref[...], preferred_element_type=jnp.float32)
        k = jnp.dot(x, wk_ref[...], preferred_element_type=jnp.float32)
        v = jnp.dot(x, wv_ref[...], preferred_element_type=jnp.float32)
        rows = pl.ds(i * t, t)
        qkv_ref[rows, 0:d] = (q + b[:, 0:d]).astype(jnp.bfloat16)
        qkv_ref[rows, d:2 * d] = (k + b[:, d:2 * d]).astype(jnp.bfloat16)
        qkv_ref[rows, 2 * d:3 * d] = (v + b[:, 2 * d:3 * d]).astype(jnp.bfloat16)

    @pl.when(i >= n_tiles)
    def _():
        q = qkv_ref[pl.ds((i - n_tiles) * t, t), 0:d]
        k = qkv_ref[:, d:2 * d]
        v = qkv_ref[:, 2 * d:3 * d]
        s = jax.lax.dot_general(q, k, (((1,), (1,)), ((), ())),
                                preferred_element_type=jnp.float32) * scale
        m = jnp.max(s, axis=1, keepdims=True)
        p = jnp.exp(s - m)
        p = p / jnp.sum(p, axis=1, keepdims=True)
        attn = jnp.dot(p.astype(jnp.bfloat16), v,
                       preferred_element_type=jnp.float32)
        o = jnp.dot(attn, wo_ref[...],
                    preferred_element_type=jnp.float32) + bo_ref[...]
        xa_ref[...] = x_ref[...] + o


def _moe_kernel(xa_ref, wr_ref, w1_ref, b1_ref, w2_ref, b2_ref, out_ref,
                gates_ref, *, n_chunks):
    s = pl.program_id(1)
    e = s // 2
    half = s % 2

    @pl.when(s == 0)
    def _():
        xa = xa_ref[...]
        logits = jnp.dot(xa, wr_ref[...], preferred_element_type=jnp.float32)
        lmax = jnp.max(logits, axis=1, keepdims=True)
        ex = jnp.exp(logits - lmax)
        gates_ref[...] = ex / jnp.sum(ex, axis=1, keepdims=True)
        out_ref[...] = xa

    gates = gates_ref[...]
    onehot = (jax.lax.broadcasted_iota(jnp.int32, (1, gates.shape[1]), 1) == e)
    g = jnp.sum(gates * onehot, axis=1, keepdims=True)

    xa = xa_ref[...]
    fc = w1_ref.shape[2] // n_chunks
    part = None
    for c in range(n_chunks):
        sl = slice(c * fc, (c + 1) * fc)
        h = jnp.dot(xa, w1_ref[0, :, sl], preferred_element_type=jnp.float32)
        h = jnp.maximum(h + b1_ref[0, :, sl], 0.0)
        p = jnp.dot(h, w2_ref[0, sl, :], preferred_element_type=jnp.float32)
        part = p if part is None else part + p
    b2term = b2_ref[0] * (half == 0).astype(jnp.float32)
    out_ref[...] += g * (part + b2term)


def kernel(x, Wq, bq, Wk, bk, Wv, bv, Wo, bo, Wr, W1, b1, W2, b2):
    B, S, D = x.shape
    E, _, F = W1.shape
    xf = x.reshape(S, D)
    scale = 1.0 / math.sqrt(D)

    bqkv = jnp.concatenate([bq, bk, bv]).reshape(1, 3 * D)
    Ta = min(512, S)
    n_tiles = S // Ta
    xtile = lambda i: (jnp.where(i < n_tiles, i, i - n_tiles), 0)
    xa = pl.pallas_call(
        functools.partial(_attn_fused_kernel, n_tiles=n_tiles, scale=scale),
        grid=(2 * n_tiles,),
        in_specs=[
            pl.BlockSpec((Ta, D), xtile),
            pl.BlockSpec((D, D), lambda i: (0, 0)),
            pl.BlockSpec((D, D), lambda i: (0, 0)),
            pl.BlockSpec((D, D), lambda i: (0, 0)),
            pl.BlockSpec((1, 3 * D), lambda i: (0, 0)),
            pl.BlockSpec((D, D), lambda i: (0, 0)),
            pl.BlockSpec((1, D), lambda i: (0, 0)),
        ],
        out_specs=pl.BlockSpec((Ta, D), xtile),
        out_shape=jax.ShapeDtypeStruct((S, D), jnp.float32),
        scratch_shapes=[
            pltpu.VMEM((S, 3 * D), jnp.bfloat16),
        ],
        compiler_params=pltpu.CompilerParams(
            dimension_semantics=("arbitrary",)),
    )(xf, Wq, Wk, Wv, bqkv, Wo, bo.reshape(1, D))

    Tm = min(2048, S)
    b1r = b1.reshape(E, 1, F)
    b2r = b2.reshape(E, 1, D)
    out = pl.pallas_call(
        functools.partial(_moe_kernel, n_chunks=2),
        grid=(S // Tm, 2 * E),
        in_specs=[
            pl.BlockSpec((Tm, D), lambda t, s: (t, 0)),
            pl.BlockSpec((D, E), lambda t, s: (0, 0)),
            pl.BlockSpec((1, D, F // 2),
                         lambda t, s: (s // 2, 0, s % 2)),
            pl.BlockSpec((1, 1, F // 2),
                         lambda t, s: (s // 2, 0, s % 2)),
            pl.BlockSpec((1, F // 2, D),
                         lambda t, s: (s // 2, s % 2, 0)),
            pl.BlockSpec((1, 1, D), lambda t, s: (s // 2, 0, 0)),
        ],
        out_specs=pl.BlockSpec((Tm, D), lambda t, s: (t, 0)),
        out_shape=jax.ShapeDtypeStruct((S, D), jnp.float32),
        scratch_shapes=[
            pltpu.VMEM((Tm, E), jnp.float32),
        ],
        compiler_params=pltpu.CompilerParams(
            dimension_semantics=("parallel", "arbitrary")),
    )(xa, Wr, W1, b1r, W2, b2r)

    return out.reshape(B, S, D)
